# baseline (speedup 1.0000x reference)
import jax
import jax.numpy as jnp
from jax import lax
from jax.experimental import pallas as pl
from jax.experimental.pallas import tpu as pltpu
from jax.experimental.pallas import tpu_sc as plsc

N = 100000
E = 3200000
G = 128
NC = 2
NS = 16
NW = NC * NS
BK = 2048
NB = 49
NP = NB * BK
EP = NW * NP
F32 = jnp.float32
NEG_INF = float("-inf")


def _mesh():
    return plsc.VectorSubcoreMesh(
        core_axis_name="c", subcore_axis_name="s", num_cores=NC, num_subcores=NS
    )


_SC_PARAMS_NL = pltpu.CompilerParams(
    use_tc_tiling_on_sc=False, needs_layout_passes=False)


def _deg_body(dst_hbm, zeros_hbm, out_hbm, idxv, accum):
    c = lax.axis_index("c")
    s = lax.axis_index("s")
    w = c * NS + s
    pltpu.sync_copy(zeros_hbm, accum)
    ept = EP // NW
    base = w * ept
    ones16 = jnp.ones((16,), F32)

    def chunk(i, carry):
        off = pl.multiple_of(base + i * 512, 8)
        pltpu.sync_copy(dst_hbm.at[pl.ds(off, 512)], idxv)

        def inner(j, carry2):
            v = idxv[pl.ds(j * 16, 16)]
            plsc.addupdate_scatter(accum, [v], ones16)
            return carry2

        return lax.fori_loop(0, 32, inner, carry)

    lax.fori_loop(0, ept // 512, chunk, 0)
    pltpu.sync_copy(accum, out_hbm.at[w])


def _tc0_body(d_ref, o_ref):
    o_ref[...] = jnp.sum(d_ref[...], axis=0, keepdims=True)[None]


def _tc1_body(d_ref, x_ref, dinv_ref, y1_ref):
    deg = d_ref[...] + 1.0
    dv = lax.rsqrt(deg)
    dinv_ref[...] = dv
    y1_ref[...] = x_ref[...] * dv


def _tc2_body(p_ref, y1_ref, dinv_ref, w_ref, b_ref, y2_ref):
    dv = dinv_ref[...]
    sacc = (p_ref[0] + p_ref[1] + y1_ref[...]) * dv
    h1 = jnp.dot(sacc, w_ref[...], preferred_element_type=F32) + b_ref[...]
    y2 = jnp.maximum(h1, 0.0) * dv
    y2_ref[0] = y2[:, 0:8]
    y2_ref[1] = y2[:, 8:16]
    y2_ref[2] = y2[:, 16:24]
    y2_ref[3] = y2[:, 24:32]


def _tc3_body(starts_ref, ends_ref, q_ref, y2_ref, dinv_ref, batch_ref,
              w_ref, b_ref, out_ref):
    b = pl.program_id(0)

    @pl.when(b == 0)
    def _():
        out_ref[...] = jnp.full((G, 64), NEG_INF, F32)

    dv = dinv_ref[...]
    s2 = jnp.concatenate(
        [q_ref[i] + y2_ref[i] for i in range(4)], axis=1) * dv
    h2 = jnp.dot(s2, w_ref[...], preferred_element_type=F32) + b_ref[...]
    h2 = jnp.maximum(h2, 0.0)
    r = b * BK + lax.broadcasted_iota(jnp.int32, (BK, 1), 0)
    h2 = jnp.where(r < N, h2, NEG_INF)
    ids = batch_ref[...]
    lo = starts_ref[b]
    hi = ends_ref[b]

    def seg(g, carry):
        v = jnp.max(jnp.where(ids == g, h2, NEG_INF), axis=0, keepdims=True)
        out_ref[pl.ds(g, 1), :] = jnp.maximum(out_ref[pl.ds(g, 1), :], v)
        return carry

    lax.fori_loop(lo, hi + 1, seg, 0)


def _tc4_body(p_ref, w1_ref, b1_ref, w2_ref, b2_ref, out_ref):
    h = jnp.dot(p_ref[...], w1_ref[...], preferred_element_type=F32) + b1_ref[...]
    h = jnp.maximum(h, 0.0)
    out_ref[...] = jnp.dot(h, w2_ref[...], preferred_element_type=F32) + b2_ref[...]


def kernel(x, edge_index, batch, W1, b1, W2, b2, Wf1, bf1, Wf2, bf2):
    f32 = F32
    pad_e = EP - E
    srcp = jnp.concatenate([edge_index[0], jnp.full((pad_e,), N, jnp.int32)])
    dstp = jnp.concatenate([edge_index[1], jnp.full((pad_e,), N, jnp.int32)])
    xp = jnp.pad(x, ((0, NP - N), (0, 1)))
    batchp = jnp.pad(batch, (0, NP - N), constant_values=G - 1).reshape(NP, 1)
    bidx = jnp.arange(NB, dtype=jnp.int32)
    starts = batch[jnp.minimum(bidx * BK, N - 1)]
    ends = batch[jnp.minimum((bidx + 1) * BK - 1, N - 1)]
    W1p = jnp.pad(W1, ((0, 1), (0, 0)))
    zdeg = jnp.zeros((NP,), f32)

    mesh = _mesh()

    degp = pl.kernel(
        _deg_body,
        out_type=jax.ShapeDtypeStruct((NW, NP), f32),
        mesh=mesh,
        compiler_params=_SC_PARAMS_NL,
        scratch_types=[
            pltpu.VMEM((512,), jnp.int32),
            pltpu.VMEM((NP,), f32),
        ],
    )(dstp, zdeg)

    degrows = pl.pallas_call(
        _tc0_body,
        grid=(NB,),
        in_specs=[pl.BlockSpec((NW, BK), lambda b: (0, b))],
        out_specs=pl.BlockSpec((1, 1, BK), lambda b: (b, 0, 0)),
        out_shape=jax.ShapeDtypeStruct((NB, 1, BK), f32),
    )(degp.reshape(NW, NP))
    deg_col = degrows.reshape(NP, 1)

    dinv, y1 = pl.pallas_call(
        _tc1_body,
        grid=(NB,),
        in_specs=[
            pl.BlockSpec((BK, 1), lambda b: (b, 0)),
            pl.BlockSpec((BK, 4), lambda b: (b, 0)),
        ],
        out_specs=[
            pl.BlockSpec((BK, 1), lambda b: (b, 0)),
            pl.BlockSpec((BK, 4), lambda b: (b, 0)),
        ],
        out_shape=[
            jax.ShapeDtypeStruct((NP, 1), f32),
            jax.ShapeDtypeStruct((NP, 4), f32),
        ],
    )(deg_col, xp)

    half = EP // NC
    p1 = jnp.stack([
        jnp.zeros((NP, 4), f32).at[dstp[c*half:(c+1)*half]].add(
            y1[srcp[c*half:(c+1)*half]]) for c in range(NC)])

    y2s = pl.pallas_call(
        _tc2_body,
        grid=(NB,),
        in_specs=[
            pl.BlockSpec((NC, BK, 4), lambda b: (0, b, 0)),
            pl.BlockSpec((BK, 4), lambda b: (b, 0)),
            pl.BlockSpec((BK, 1), lambda b: (b, 0)),
            pl.BlockSpec((4, 32), lambda b: (0, 0)),
            pl.BlockSpec((1, 32), lambda b: (0, 0)),
        ],
        out_specs=pl.BlockSpec((4, BK, 8), lambda b: (0, b, 0)),
        out_shape=jax.ShapeDtypeStruct((4, NP, 8), f32),
    )(p1, y1, dinv, W1p, b1.reshape(1, 32))

    table2 = y2s.reshape(4 * NP, 8)
    q2 = jnp.stack([
        jnp.zeros((NP, 8), f32).at[dstp].add(table2[srcp + q * NP])
        for q in range(4)])

    pooled = pl.pallas_call(
        _tc3_body,
        grid_spec=pltpu.PrefetchScalarGridSpec(
            num_scalar_prefetch=2,
            grid=(NB,),
            in_specs=[
                pl.BlockSpec((4, BK, 8), lambda b, s0, s1: (0, b, 0)),
                pl.BlockSpec((4, BK, 8), lambda b, s0, s1: (0, b, 0)),
                pl.BlockSpec((BK, 1), lambda b, s0, s1: (b, 0)),
                pl.BlockSpec((BK, 1), lambda b, s0, s1: (b, 0)),
                pl.BlockSpec((32, 64), lambda b, s0, s1: (0, 0)),
                pl.BlockSpec((1, 64), lambda b, s0, s1: (0, 0)),
            ],
            out_specs=pl.BlockSpec((G, 64), lambda b, s0, s1: (0, 0)),
        ),
        out_shape=jax.ShapeDtypeStruct((G, 64), f32),
    )(starts, ends, q2, y2s, dinv, batchp, W2, b2.reshape(1, 64))

    out = pl.pallas_call(
        _tc4_body,
        out_shape=jax.ShapeDtypeStruct((G, 10), f32),
    )(pooled, Wf1, bf1.reshape(1, 512), Wf2, bf2.reshape(1, 10))
    return out


# speedup vs baseline: 6.6669x; 6.6669x over previous
import jax
import jax.numpy as jnp
from jax import lax
from jax.experimental import pallas as pl
from jax.experimental.pallas import tpu as pltpu
from jax.experimental.pallas import tpu_sc as plsc

N = 100000
E = 3200000
G = 128
NC = 2
NS = 16
NW = NC * NS
BK = 2048
NB = 49
NP = NB * BK
EP = NW * NP
F32 = jnp.float32
NEG_INF = float("-inf")


def _mesh():
    return plsc.VectorSubcoreMesh(
        core_axis_name="c", subcore_axis_name="s", num_cores=NC, num_subcores=NS
    )


_SC_PARAMS_NL = pltpu.CompilerParams(
    use_tc_tiling_on_sc=False, needs_layout_passes=False)


def _deg_body(dst_hbm, zeros_hbm, out_hbm, idxv, accum):
    c = lax.axis_index("c")
    s = lax.axis_index("s")
    w = c * NS + s
    pltpu.sync_copy(zeros_hbm, accum)
    ept = EP // NW
    base = w * ept
    ones16 = jnp.ones((16,), F32)

    def chunk(i, carry):
        off = pl.multiple_of(base + i * 512, 8)
        pltpu.sync_copy(dst_hbm.at[pl.ds(off, 512)], idxv)

        def inner(j, carry2):
            v = idxv[pl.ds(j * 16, 16)]
            plsc.addupdate_scatter(accum, [v], ones16)
            return carry2

        return lax.fori_loop(0, 32, inner, carry)

    lax.fori_loop(0, ept // 512, chunk, 0)
    pltpu.sync_copy(accum, out_hbm.at[w])


def _tc0_body(d_ref, o_ref):
    o_ref[...] = jnp.sum(d_ref[...], axis=0, keepdims=True)[None]


def _tc1_body(d_ref, x_ref, dinv_ref, y1_ref):
    deg = d_ref[...] + 1.0
    dv = lax.rsqrt(deg)
    dinv_ref[...] = dv
    y1_ref[...] = x_ref[...] * dv


def _tc2_body(p_ref, y1_ref, dinv_ref, w_ref, b_ref, y2_ref):
    dv = dinv_ref[...]
    sacc = (p_ref[...] + y1_ref[...]) * dv
    h1 = jnp.dot(sacc, w_ref[...], preferred_element_type=F32) + b_ref[...]
    y2_ref[...] = jnp.maximum(h1, 0.0) * dv


def _tc3_body(starts_ref, ends_ref, q_ref, y2_ref, dinv_ref, batch_ref,
              w_ref, b_ref, out_ref):
    b = pl.program_id(0)

    @pl.when(b == 0)
    def _():
        out_ref[...] = jnp.full((G, 64), NEG_INF, F32)

    dv = dinv_ref[...]
    s2 = (q_ref[...] + y2_ref[...]) * dv
    h2 = jnp.dot(s2, w_ref[...], preferred_element_type=F32) + b_ref[...]
    h2 = jnp.maximum(h2, 0.0)
    r = b * BK + lax.broadcasted_iota(jnp.int32, (BK, 1), 0)
    h2 = jnp.where(r < N, h2, NEG_INF)
    ids = batch_ref[...]
    lo = starts_ref[b]
    hi = ends_ref[b]

    def seg(g, carry):
        v = jnp.max(jnp.where(ids == g, h2, NEG_INF), axis=0, keepdims=True)
        out_ref[pl.ds(g, 1), :] = jnp.maximum(out_ref[pl.ds(g, 1), :], v)
        return carry

    lax.fori_loop(lo, hi + 1, seg, 0)


def _tc4_body(p_ref, w1_ref, b1_ref, w2_ref, b2_ref, out_ref):
    h = jnp.dot(p_ref[...], w1_ref[...], preferred_element_type=F32) + b1_ref[...]
    h = jnp.maximum(h, 0.0)
    out_ref[...] = jnp.dot(h, w2_ref[...], preferred_element_type=F32) + b2_ref[...]


def kernel(x, edge_index, batch, W1, b1, W2, b2, Wf1, bf1, Wf2, bf2):
    f32 = F32
    pad_e = EP - E
    srcp = jnp.concatenate([edge_index[0], jnp.full((pad_e,), N, jnp.int32)])
    dstp = jnp.concatenate([edge_index[1], jnp.full((pad_e,), N, jnp.int32)])
    xp = jnp.pad(x, ((0, NP - N), (0, 1)))
    batchp = jnp.pad(batch, (0, NP - N), constant_values=G - 1).reshape(NP, 1)
    bidx = jnp.arange(NB, dtype=jnp.int32)
    starts = batch[jnp.minimum(bidx * BK, N - 1)]
    ends = batch[jnp.minimum((bidx + 1) * BK - 1, N - 1)]
    W1p = jnp.pad(W1, ((0, 1), (0, 0)))
    zdeg = jnp.zeros((NP,), f32)

    mesh = _mesh()

    degp = pl.kernel(
        _deg_body,
        out_type=jax.ShapeDtypeStruct((NW, NP), f32),
        mesh=mesh,
        compiler_params=_SC_PARAMS_NL,
        scratch_types=[
            pltpu.VMEM((512,), jnp.int32),
            pltpu.VMEM((NP,), f32),
        ],
    )(dstp, zdeg)

    degrows = pl.pallas_call(
        _tc0_body,
        grid=(NB,),
        in_specs=[pl.BlockSpec((NW, BK), lambda b: (0, b))],
        out_specs=pl.BlockSpec((1, 1, BK), lambda b: (b, 0, 0)),
        out_shape=jax.ShapeDtypeStruct((NB, 1, BK), f32),
    )(degp.reshape(NW, NP))
    deg_col = degrows.reshape(NP, 1)

    dinv, y1 = pl.pallas_call(
        _tc1_body,
        grid=(NB,),
        in_specs=[
            pl.BlockSpec((BK, 1), lambda b: (b, 0)),
            pl.BlockSpec((BK, 4), lambda b: (b, 0)),
        ],
        out_specs=[
            pl.BlockSpec((BK, 1), lambda b: (b, 0)),
            pl.BlockSpec((BK, 4), lambda b: (b, 0)),
        ],
        out_shape=[
            jax.ShapeDtypeStruct((NP, 1), f32),
            jax.ShapeDtypeStruct((NP, 4), f32),
        ],
    )(deg_col, xp)

    p1 = jnp.zeros((NP, 4), f32).at[dstp].add(y1[srcp])

    y2s = pl.pallas_call(
        _tc2_body,
        grid=(NB,),
        in_specs=[
            pl.BlockSpec((BK, 4), lambda b: (b, 0)),
            pl.BlockSpec((BK, 4), lambda b: (b, 0)),
            pl.BlockSpec((BK, 1), lambda b: (b, 0)),
            pl.BlockSpec((4, 32), lambda b: (0, 0)),
            pl.BlockSpec((1, 32), lambda b: (0, 0)),
        ],
        out_specs=pl.BlockSpec((BK, 32), lambda b: (b, 0)),
        out_shape=jax.ShapeDtypeStruct((NP, 32), f32),
    )(p1, y1, dinv, W1p, b1.reshape(1, 32))

    q2 = jnp.zeros((NP, 32), f32).at[dstp].add(y2s[srcp])

    pooled = pl.pallas_call(
        _tc3_body,
        grid_spec=pltpu.PrefetchScalarGridSpec(
            num_scalar_prefetch=2,
            grid=(NB,),
            in_specs=[
                pl.BlockSpec((BK, 32), lambda b, s0, s1: (b, 0)),
                pl.BlockSpec((BK, 32), lambda b, s0, s1: (b, 0)),
                pl.BlockSpec((BK, 1), lambda b, s0, s1: (b, 0)),
                pl.BlockSpec((BK, 1), lambda b, s0, s1: (b, 0)),
                pl.BlockSpec((32, 64), lambda b, s0, s1: (0, 0)),
                pl.BlockSpec((1, 64), lambda b, s0, s1: (0, 0)),
            ],
            out_specs=pl.BlockSpec((G, 64), lambda b, s0, s1: (0, 0)),
        ),
        out_shape=jax.ShapeDtypeStruct((G, 64), f32),
    )(starts, ends, q2, y2s, dinv, batchp, W2, b2.reshape(1, 64))

    out = pl.pallas_call(
        _tc4_body,
        out_shape=jax.ShapeDtypeStruct((G, 10), f32),
    )(pooled, Wf1, bf1.reshape(1, 512), Wf2, bf2.reshape(1, 10))
    return out
